# Initial kernel scaffold; baseline (speedup 1.0000x reference)
import jax
import jax.numpy as jnp
from jax.experimental import pallas as pl


def kernel(x, table):
    raise NotImplementedError("write your pallas kernel here")


# speedup vs baseline: 4.7512x; 4.7512x over previous
import functools

import jax
import jax.numpy as jnp
from jax import lax
from jax.experimental import pallas as pl
from jax.experimental.pallas import tpu as pltpu
from jax.experimental.pallas import tpu_sc as plsc

_NC = 2
_NS = 16
_NW = _NC * _NS

_IW = 128
_K = 4


@functools.cache
def _build(n_rows, vocab, d, dtype):
    rows_per_w = n_rows // _NW
    n_steps = rows_per_w // _K
    chunk = _K * _IW

    mesh = plsc.VectorSubcoreMesh(
        core_axis_name="c", subcore_axis_name="s",
        num_cores=_NC, num_subcores=_NS,
    )

    @functools.partial(
        pl.kernel,
        out_type=jax.ShapeDtypeStruct((n_rows * _IW, d), dtype),
        mesh=mesh,
        scratch_types=[
            pltpu.VMEM((_K, _IW), jnp.int32),
            pltpu.VMEM((chunk, d), dtype),
            pltpu.SemaphoreType.DMA,
        ],
        compiler_params=pltpu.CompilerParams(use_tc_tiling_on_sc=False),
    )
    def gather(idx_hbm, table_hbm, out_hbm, idx_v, rows_v, sem):
        wid = lax.axis_index("s") * _NC + lax.axis_index("c")
        row0 = wid * rows_per_w

        @pl.loop(0, n_steps)
        def _step(g):
            r = row0 + g * _K
            pltpu.sync_copy(idx_hbm.at[pl.ds(r, _K)], idx_v)
            copies = [
                pltpu.async_copy(
                    table_hbm.at[idx_v.at[j]],
                    rows_v.at[pl.ds(j * _IW, _IW)],
                    sem,
                )
                for j in range(_K)
            ]
            for c in copies:
                c.wait()
            pltpu.sync_copy(rows_v, out_hbm.at[pl.ds(r * _IW, chunk)])

    return gather


def kernel(x, table):
    b, h = x.shape
    vocab, d = table.shape
    flat = x.reshape(-1).astype(jnp.int32)
    n = flat.shape[0]
    assert n % (_NW * _K * _IW) == 0
    idx2d = flat.reshape(n // _IW, _IW)
    out = _build(n // _IW, vocab, d, table.dtype)(idx2d, table)
    return out.reshape(b, h, d)


# speedup vs baseline: 5.1631x; 1.0867x over previous
import functools

import jax
import jax.numpy as jnp
from jax import lax
from jax.experimental import pallas as pl
from jax.experimental.pallas import tpu as pltpu
from jax.experimental.pallas import tpu_sc as plsc

_NC = 2
_NS = 16
_NW = _NC * _NS

_IW = 128
_K = 4
_NBUF = 2


@functools.cache
def _build(n_rows, vocab, d, dtype):
    rows_per_w = n_rows // _NW
    n_steps = rows_per_w // _K
    chunk = _K * _IW

    mesh = plsc.VectorSubcoreMesh(
        core_axis_name="c", subcore_axis_name="s",
        num_cores=_NC, num_subcores=_NS,
    )

    @functools.partial(
        pl.kernel,
        out_type=jax.ShapeDtypeStruct((n_rows * _IW, d), dtype),
        mesh=mesh,
        scratch_types=[
            pltpu.VMEM((_NBUF, _K, _IW), jnp.int32),
            pltpu.VMEM((_NBUF, chunk, d), dtype),
            [pltpu.SemaphoreType.DMA] * _NBUF,
            [pltpu.SemaphoreType.DMA] * _NBUF,
            [pltpu.SemaphoreType.DMA] * _NBUF,
        ],
        compiler_params=pltpu.CompilerParams(use_tc_tiling_on_sc=False),
    )
    def gather(idx_hbm, table_hbm, out_hbm, idx_v, rows_v, isems, gsems, osems):
        wid = lax.axis_index("s") * _NC + lax.axis_index("c")
        row0 = wid * rows_per_w

        for b in range(_NBUF):
            pltpu.async_copy(
                idx_hbm.at[pl.ds(row0 + b * _K, _K)], idx_v.at[b], isems[b])

        @pl.loop(0, n_steps, step=_NBUF)
        def _step(g0):
            for b in range(_NBUF):
                g = g0 + b
                r = row0 + g * _K
                pltpu.make_async_copy(
                    idx_hbm.at[pl.ds(r, _K)], idx_v.at[b], isems[b]).wait()

                @pl.when(g0 >= _NBUF)
                def _():
                    pltpu.make_async_copy(
                        rows_v.at[b], out_hbm.at[pl.ds(r * _IW, chunk)],
                        osems[b]).wait()

                copies = [
                    pltpu.async_copy(
                        table_hbm.at[idx_v.at[b].at[j]],
                        rows_v.at[b].at[pl.ds(j * _IW, _IW)],
                        gsems[b],
                    )
                    for j in range(_K)
                ]
                for c in copies:
                    c.wait()

                pltpu.async_copy(
                    rows_v.at[b], out_hbm.at[pl.ds(r * _IW, chunk)], osems[b])

                @pl.when(g + _NBUF < n_steps)
                def _():
                    pltpu.async_copy(
                        idx_hbm.at[pl.ds(r + _NBUF * _K, _K)], idx_v.at[b],
                        isems[b])

        for b in range(_NBUF):
            pltpu.make_async_copy(
                rows_v.at[b], out_hbm.at[pl.ds(row0 * _IW, chunk)],
                osems[b]).wait()

    return gather


def kernel(x, table):
    b, h = x.shape
    vocab, d = table.shape
    flat = x.reshape(-1).astype(jnp.int32)
    n = flat.shape[0]
    assert n % (_NW * _K * _IW) == 0
    idx2d = flat.reshape(n // _IW, _IW)
    out = _build(n // _IW, vocab, d, table.dtype)(idx2d, table)
    return out.reshape(b, h, d)


# speedup vs baseline: 5.1809x; 1.0035x over previous
import functools

import jax
import jax.numpy as jnp
from jax import lax
from jax.experimental import pallas as pl
from jax.experimental.pallas import tpu as pltpu
from jax.experimental.pallas import tpu_sc as plsc

_NC = 2
_NS = 16
_NW = _NC * _NS

_IW = 128
_K = 5
_NBUF = 2


@functools.cache
def _build(n_rows, vocab, d, dtype):
    rows_per_w = n_rows // _NW
    n_steps = rows_per_w // _K
    assert n_steps % _NBUF == 0
    chunk = _K * _IW

    mesh = plsc.VectorSubcoreMesh(
        core_axis_name="c", subcore_axis_name="s",
        num_cores=_NC, num_subcores=_NS,
    )

    @functools.partial(
        pl.kernel,
        out_type=jax.ShapeDtypeStruct((n_rows * _IW, d), dtype),
        mesh=mesh,
        scratch_types=[
            pltpu.VMEM((_NBUF, _K, _IW), jnp.int32),
            pltpu.VMEM((_NBUF, chunk, d), dtype),
            [pltpu.SemaphoreType.DMA] * _NBUF,
            [pltpu.SemaphoreType.DMA] * _NBUF,
            [pltpu.SemaphoreType.DMA] * _NBUF,
        ],
        compiler_params=pltpu.CompilerParams(use_tc_tiling_on_sc=False),
    )
    def gather(idx_hbm, table_hbm, out_hbm, idx_v, rows_v, isems, gsems, osems):
        wid = lax.axis_index("s") * _NC + lax.axis_index("c")
        row0 = wid * rows_per_w

        def fire_gathers(b):
            for j in range(_K):
                pltpu.async_copy(
                    table_hbm.at[idx_v.at[b].at[j]],
                    rows_v.at[b].at[pl.ds(j * _IW, _IW)],
                    gsems[b],
                )

        def wait_gathers(b):
            for j in range(_K):
                pltpu.make_async_copy(
                    table_hbm.at[idx_v.at[b].at[j]],
                    rows_v.at[b].at[pl.ds(j * _IW, _IW)],
                    gsems[b],
                ).wait()

        for b in range(_NBUF):
            pltpu.async_copy(
                idx_hbm.at[pl.ds(row0 + b * _K, _K)], idx_v.at[b], isems[b])

        @pl.loop(0, n_steps, step=_NBUF)
        def _step(g0):
            for b in range(_NBUF):
                g = g0 + b
                r = row0 + g * _K
                p = (b - 1) % _NBUF

                @pl.when(g0 >= _NBUF)
                def _():
                    pltpu.make_async_copy(
                        rows_v.at[b], out_hbm.at[pl.ds(r * _IW, chunk)],
                        osems[b]).wait()

                pltpu.make_async_copy(
                    idx_hbm.at[pl.ds(r, _K)], idx_v.at[b], isems[b]).wait()

                fire_gathers(b)

                @pl.when(g >= 1)
                def _():
                    wait_gathers(p)
                    pltpu.async_copy(
                        rows_v.at[p],
                        out_hbm.at[pl.ds((r - _K) * _IW, chunk)], osems[p])

                    @pl.when(g - 1 + _NBUF < n_steps)
                    def _():
                        pltpu.async_copy(
                            idx_hbm.at[pl.ds(r - _K + _NBUF * _K, _K)],
                            idx_v.at[p], isems[p])

        last = (n_steps - 1) % _NBUF
        r_last = row0 + (n_steps - 1) * _K
        wait_gathers(last)
        pltpu.async_copy(
            rows_v.at[last], out_hbm.at[pl.ds(r_last * _IW, chunk)],
            osems[last])
        for b in range(_NBUF):
            pltpu.make_async_copy(
                rows_v.at[b], out_hbm.at[pl.ds(row0 * _IW, chunk)],
                osems[b]).wait()

    return gather


def kernel(x, table):
    b, h = x.shape
    vocab, d = table.shape
    flat = x.reshape(-1).astype(jnp.int32)
    n = flat.shape[0]
    assert n % (_NW * _K * _IW) == 0
    idx2d = flat.reshape(n // _IW, _IW)
    out = _build(n // _IW, vocab, d, table.dtype)(idx2d, table)
    return out.reshape(b, h, d)
